# Initial kernel scaffold; baseline (speedup 1.0000x reference)
import functools

import jax
import jax.numpy as jnp
from jax import lax
from jax.experimental import pallas as pl
from jax.experimental.pallas import tpu as pltpu
from jax.experimental.pallas import tpu_sc as plsc

NUM_SEGMENTS = 16384
NUM_ATOM_TYPES = 100
HIDDEN = 64
N_ATOMS = 800000

NUM_CORES = 2
NUM_SUBCORES = 16
NW = NUM_CORES * NUM_SUBCORES

CHUNK = 128
CPW = 196
APW = CPW * CHUNK
N_PADDED = NW * APW

HIST_ROWS = NUM_SEGMENTS * NUM_ATOM_TYPES
HIST_ROWS_PAD = HIST_ROWS + CHUNK
ROWS_PER_TILE = HIST_ROWS // NUM_SUBCORES
STAGE = 12800
STAGE_PER_TILE = ROWS_PER_TILE // STAGE


def _sc_hist_body(zp_hbm, bp_hbm, out_hbm, zbuf, bbuf, idxbuf, ones_v,
                  stage, hist):
    c = lax.axis_index("c")
    s = lax.axis_index("s")
    wid = c * NUM_SUBCORES + s

    @pl.loop(0, STAGE // 16)
    def _zero_fill(i):
        stage[pl.ds(i * 16, 16)] = jnp.zeros((16,), jnp.float32)

    for k in range(STAGE_PER_TILE):
        pltpu.sync_copy(stage, hist.at[pl.ds(s * ROWS_PER_TILE + k * STAGE,
                                             STAGE)])

    @pl.when(s == 0)
    def _zero_dummy():
        pltpu.sync_copy(stage.at[pl.ds(0, CHUNK)],
                        hist.at[pl.ds(HIST_ROWS, CHUNK)])

    plsc.subcore_barrier()

    base = wid * APW
    pltpu.sync_copy(zp_hbm.at[pl.ds(base, APW)], zbuf)
    pltpu.sync_copy(bp_hbm.at[pl.ds(base, APW)], bbuf)

    for i in range(CHUNK // 16):
        ones_v[pl.ds(i * 16, 16)] = jnp.ones((16,), jnp.float32)

    @pl.loop(0, CPW)
    def _scatter(j):
        for k in range(CHUNK // 16):
            off = j * CHUNK + k * 16
            vb = bbuf[pl.ds(off, 16)]
            vz = zbuf[pl.ds(off, 16)]
            idxbuf[j, pl.ds(k * 16, 16)] = vb * NUM_ATOM_TYPES + vz
        pltpu.sync_copy(ones_v, hist.at[idxbuf.at[j]], add=True)

    plsc.subcore_barrier()

    for k in range(STAGE_PER_TILE):
        off = s * ROWS_PER_TILE + k * STAGE
        pltpu.sync_copy(hist.at[pl.ds(off, STAGE)], stage)
        pltpu.sync_copy(stage, out_hbm.at[c, pl.ds(off, STAGE)])


_sc_hist = functools.partial(
    pl.kernel,
    out_type=jax.ShapeDtypeStruct((NUM_CORES, HIST_ROWS), jnp.float32),
    mesh=plsc.VectorSubcoreMesh(core_axis_name="c", subcore_axis_name="s"),
    scratch_types=[
        pltpu.VMEM((APW,), jnp.int32),
        pltpu.VMEM((APW,), jnp.int32),
        pltpu.VMEM((CPW, CHUNK), jnp.int32),
        pltpu.VMEM((CHUNK,), jnp.float32),
        pltpu.VMEM((STAGE,), jnp.float32),
        pltpu.VMEM_SHARED((HIST_ROWS_PAD,), jnp.float32),
    ],
)(_sc_hist_body)


SEG_BLK = 1024


def _mlp_body(h0_ref, h1_ref, emb_ref, w1_ref, b1_ref, w2_ref, b2_ref,
              out_ref):
    hist = h0_ref[...] + h1_ref[...]
    counts = jnp.sum(hist, axis=1, keepdims=True)
    sums = jnp.dot(hist, emb_ref[...], preferred_element_type=jnp.float32)
    pooled = sums / jnp.maximum(counts, 1.0)
    h = jnp.maximum(
        jnp.dot(pooled, w1_ref[...], preferred_element_type=jnp.float32)
        + b1_ref[...], 0.0)
    out_ref[...] = (
        jnp.dot(h, w2_ref[...], preferred_element_type=jnp.float32)
        + b2_ref[...])


def _mlp(h0, h1, emb, W1, b1, W2, b2):
    return pl.pallas_call(
        _mlp_body,
        grid=(NUM_SEGMENTS // SEG_BLK,),
        in_specs=[
            pl.BlockSpec((SEG_BLK, NUM_ATOM_TYPES), lambda i: (i, 0)),
            pl.BlockSpec((SEG_BLK, NUM_ATOM_TYPES), lambda i: (i, 0)),
            pl.BlockSpec((NUM_ATOM_TYPES, HIDDEN), lambda i: (0, 0)),
            pl.BlockSpec((HIDDEN, HIDDEN), lambda i: (0, 0)),
            pl.BlockSpec((1, HIDDEN), lambda i: (0, 0)),
            pl.BlockSpec((HIDDEN, 1), lambda i: (0, 0)),
            pl.BlockSpec((1, 1), lambda i: (0, 0)),
        ],
        out_specs=pl.BlockSpec((SEG_BLK, 1), lambda i: (i, 0)),
        out_shape=jax.ShapeDtypeStruct((NUM_SEGMENTS, 1), jnp.float32),
    )(h0, h1, emb, W1, b1, W2, b2)


@jax.jit
def kernel(z, batch, emb, W1, b1, W2, b2):
    pad = N_PADDED - N_ATOMS
    zp = jnp.concatenate([z.astype(jnp.int32),
                          jnp.zeros((pad,), jnp.int32)])
    bp = jnp.concatenate([batch.astype(jnp.int32),
                          jnp.full((pad,), NUM_SEGMENTS, jnp.int32)])

    hist_parts = _sc_hist(zp, bp)
    h0 = hist_parts[0].reshape(NUM_SEGMENTS, NUM_ATOM_TYPES)
    h1 = hist_parts[1].reshape(NUM_SEGMENTS, NUM_ATOM_TYPES)

    out = _mlp(h0, h1, emb, W1, b1.reshape(1, HIDDEN), W2,
               b2.reshape(1, 1))
    return out.reshape(-1)


# speedup vs baseline: 23.1232x; 23.1232x over previous
import functools

import jax
import jax.numpy as jnp
from jax import lax
from jax.experimental import pallas as pl
from jax.experimental.pallas import tpu as pltpu
from jax.experimental.pallas import tpu_sc as plsc

NUM_SEGMENTS = 16384
NUM_ATOM_TYPES = 100
HIDDEN = 64
N_ATOMS = 800000

NUM_CORES = 2
NUM_SUBCORES = 16
NW = NUM_CORES * NUM_SUBCORES

CHUNK = 128
CPW = 196
APW = CPW * CHUNK
N_PADDED = NW * APW
BCH = 28
NBATCH = CPW // BCH
BATCH_ATOMS = BCH * CHUNK

HIST_ROWS = NUM_SEGMENTS * NUM_ATOM_TYPES
HIST_ROWS_PAD = HIST_ROWS + CHUNK
ROWS_PER_TILE = HIST_ROWS // NUM_SUBCORES
STAGE = 12800
STAGE_PER_TILE = ROWS_PER_TILE // STAGE


def _sc_hist_body(zp_hbm, bp_hbm, out_hbm, zbuf, bbuf, idxbuf, ones_v,
                  stage, hist):
    c = lax.axis_index("c")
    s = lax.axis_index("s")
    wid = c * NUM_SUBCORES + s

    @pl.loop(0, STAGE // 16)
    def _zero_fill(i):
        stage[pl.ds(i * 16, 16)] = jnp.zeros((16,), jnp.float32)

    for k in range(STAGE_PER_TILE):
        pltpu.sync_copy(stage, hist.at[pl.ds(s * ROWS_PER_TILE + k * STAGE,
                                             STAGE)])

    @pl.when(s == 0)
    def _zero_dummy():
        pltpu.sync_copy(stage.at[pl.ds(0, CHUNK)],
                        hist.at[pl.ds(HIST_ROWS, CHUNK)])

    plsc.subcore_barrier()

    for i in range(CHUNK // 16):
        ones_v[pl.ds(i * 16, 16)] = jnp.ones((16,), jnp.float32)

    @pl.loop(0, NBATCH)
    def _batch(t):
        base = wid * APW + t * BATCH_ATOMS
        pltpu.sync_copy(zp_hbm.at[pl.ds(base, BATCH_ATOMS)], zbuf)
        pltpu.sync_copy(bp_hbm.at[pl.ds(base, BATCH_ATOMS)], bbuf)

        @pl.loop(0, BCH)
        def _scatter(j):
            for k in range(CHUNK // 16):
                off = j * CHUNK + k * 16
                vb = bbuf[pl.ds(off, 16)]
                vz = zbuf[pl.ds(off, 16)]
                idxbuf[j, pl.ds(k * 16, 16)] = vb * NUM_ATOM_TYPES + vz
            pltpu.sync_copy(ones_v, hist.at[idxbuf.at[j]], add=True)

    plsc.subcore_barrier()

    for k in range(STAGE_PER_TILE):
        off = s * ROWS_PER_TILE + k * STAGE
        pltpu.sync_copy(hist.at[pl.ds(off, STAGE)], stage)
        pltpu.sync_copy(stage, out_hbm.at[c, pl.ds(off, STAGE)])


@functools.cache
def _sc_hist():
  return pl.kernel(
    _sc_hist_body,
    out_type=jax.ShapeDtypeStruct((NUM_CORES, HIST_ROWS), jnp.float32),
    mesh=plsc.VectorSubcoreMesh(core_axis_name="c", subcore_axis_name="s",
                                num_cores=NUM_CORES,
                                num_subcores=NUM_SUBCORES),
    scratch_types=[
        pltpu.VMEM((BATCH_ATOMS,), jnp.int32),
        pltpu.VMEM((BATCH_ATOMS,), jnp.int32),
        pltpu.VMEM((BCH, CHUNK), jnp.int32),
        pltpu.VMEM((CHUNK,), jnp.float32),
        pltpu.VMEM((STAGE,), jnp.float32),
        pltpu.VMEM_SHARED((HIST_ROWS_PAD,), jnp.float32),
    ],
  )


SEG_BLK = 1024


def _mlp_body(h0_ref, h1_ref, emb_ref, w1_ref, b1_ref, w2_ref, b2_ref,
              out_ref):
    hist = h0_ref[...] + h1_ref[...]
    counts = jnp.sum(hist, axis=1, keepdims=True)
    sums = jnp.dot(hist, emb_ref[...], preferred_element_type=jnp.float32,
                   precision=lax.Precision.HIGHEST)
    pooled = sums / jnp.maximum(counts, 1.0)
    h = jnp.maximum(
        jnp.dot(pooled, w1_ref[...], preferred_element_type=jnp.float32,
                precision=lax.Precision.HIGHEST)
        + b1_ref[...], 0.0)
    out_ref[...] = (
        jnp.dot(h, w2_ref[...], preferred_element_type=jnp.float32,
                precision=lax.Precision.HIGHEST)
        + b2_ref[...])


def _mlp(h0, h1, emb, W1, b1, W2, b2):
    return pl.pallas_call(
        _mlp_body,
        grid=(NUM_SEGMENTS // SEG_BLK,),
        in_specs=[
            pl.BlockSpec((SEG_BLK, NUM_ATOM_TYPES), lambda i: (i, 0)),
            pl.BlockSpec((SEG_BLK, NUM_ATOM_TYPES), lambda i: (i, 0)),
            pl.BlockSpec((NUM_ATOM_TYPES, HIDDEN), lambda i: (0, 0)),
            pl.BlockSpec((HIDDEN, HIDDEN), lambda i: (0, 0)),
            pl.BlockSpec((1, HIDDEN), lambda i: (0, 0)),
            pl.BlockSpec((HIDDEN, 1), lambda i: (0, 0)),
            pl.BlockSpec((1, 1), lambda i: (0, 0)),
        ],
        out_specs=pl.BlockSpec((SEG_BLK, 1), lambda i: (i, 0)),
        out_shape=jax.ShapeDtypeStruct((NUM_SEGMENTS, 1), jnp.float32),
    )(h0, h1, emb, W1, b1, W2, b2)


@jax.jit
def kernel(z, batch, emb, W1, b1, W2, b2):
    pad = N_PADDED - N_ATOMS
    zp = jnp.concatenate([z.astype(jnp.int32),
                          jnp.zeros((pad,), jnp.int32)])
    bp = jnp.concatenate([batch.astype(jnp.int32),
                          jnp.full((pad,), NUM_SEGMENTS, jnp.int32)])

    hist_parts = _sc_hist()(zp, bp)
    h0 = hist_parts[0].reshape(NUM_SEGMENTS, NUM_ATOM_TYPES)
    h1 = hist_parts[1].reshape(NUM_SEGMENTS, NUM_ATOM_TYPES)

    out = _mlp(h0, h1, emb, W1, b1.reshape(1, HIDDEN), W2,
               b2.reshape(1, 1))
    return out.reshape(-1)


# speedup vs baseline: 27.8159x; 1.2029x over previous
import functools

import jax
import jax.numpy as jnp
from jax import lax
from jax.experimental import pallas as pl
from jax.experimental.pallas import tpu as pltpu
from jax.experimental.pallas import tpu_sc as plsc

NUM_SEGMENTS = 16384
NUM_ATOM_TYPES = 100
HIDDEN = 64
N_ATOMS = 800000

NUM_CORES = 2
NUM_SUBCORES = 16
NW = NUM_CORES * NUM_SUBCORES

SEG_PER_TILE = NUM_SEGMENTS // NW
HIST_TILE = SEG_PER_TILE * NUM_ATOM_TYPES
HIST_ROWS = NUM_SEGMENTS * NUM_ATOM_TYPES

IBUF = 12800
N_PADDED = N_ATOMS + IBUF
NBND = 48


def _sc_hist_body(idx_hbm, bnd_hbm, out_hbm, ibuf, bndvec, hist):
    c = lax.axis_index("c")
    s = lax.axis_index("s")
    wid = c * NUM_SUBCORES + s
    lo_flat = wid * HIST_TILE
    hi_flat = lo_flat + HIST_TILE

    @pl.loop(0, HIST_TILE // 16)
    def _zero(i):
        hist[pl.ds(i * 16, 16)] = jnp.zeros((16,), jnp.float32)

    pltpu.sync_copy(bnd_hbm, bndvec)
    lanes = jax.lax.broadcasted_iota(jnp.int32, (16,), 0)

    def _bnd(w):
        vec = bndvec[pl.ds((w // 16) * 16, 16)]
        onehot = jnp.int32(1) - jnp.minimum(jnp.abs(lanes - w % 16),
                                            jnp.int32(1))
        return jnp.sum(vec * onehot)

    b_lo = _bnd(wid)
    b_hi = _bnd(wid + 1)
    start0 = (b_lo // 8) * 8
    nbatch = (b_hi - start0 + IBUF - 1) // IBUF

    ones = jnp.ones((16,), jnp.float32)

    @pl.loop(0, nbatch)
    def _batch(t):
        base = start0 + t * IBUF
        pltpu.sync_copy(idx_hbm.at[pl.ds(base, IBUF)], ibuf)

        @pl.loop(0, IBUF // 16)
        def _scan(k):
            vi = ibuf[pl.ds(k * 16, 16)]
            m = (vi >= lo_flat) & (vi < hi_flat)
            vloc = jnp.where(m, vi - lo_flat, 0)
            plsc.addupdate_scatter(hist, [vloc],
                                   jnp.where(m, 1.0, 0.0).astype(jnp.float32))

    pltpu.sync_copy(hist, out_hbm.at[pl.ds(lo_flat, HIST_TILE)])


@functools.cache
def _sc_hist():
  return pl.kernel(
    _sc_hist_body,
    out_type=jax.ShapeDtypeStruct((HIST_ROWS,), jnp.float32),
    mesh=plsc.VectorSubcoreMesh(core_axis_name="c", subcore_axis_name="s",
                                num_cores=NUM_CORES,
                                num_subcores=NUM_SUBCORES),
    compiler_params=pltpu.CompilerParams(needs_layout_passes=False),
    scratch_types=[
        pltpu.VMEM((IBUF,), jnp.int32),
        pltpu.VMEM((NBND,), jnp.int32),
        pltpu.VMEM((HIST_TILE,), jnp.float32),
    ],
  )


SEG_BLK = 1024


def _mlp_body(h_ref, emb_ref, w1_ref, b1_ref, w2_ref, b2_ref, out_ref):
    hist = h_ref[...]
    counts = jnp.sum(hist, axis=1, keepdims=True)
    sums = jnp.dot(hist, emb_ref[...], preferred_element_type=jnp.float32,
                   precision=lax.Precision.HIGHEST)
    pooled = sums / jnp.maximum(counts, 1.0)
    h = jnp.maximum(
        jnp.dot(pooled, w1_ref[...], preferred_element_type=jnp.float32,
                precision=lax.Precision.HIGHEST)
        + b1_ref[...], 0.0)
    out_ref[...] = (
        jnp.dot(h, w2_ref[...], preferred_element_type=jnp.float32,
                precision=lax.Precision.HIGHEST)
        + b2_ref[...])


def _mlp(h, emb, W1, b1, W2, b2):
    return pl.pallas_call(
        _mlp_body,
        grid=(NUM_SEGMENTS // SEG_BLK,),
        in_specs=[
            pl.BlockSpec((SEG_BLK, NUM_ATOM_TYPES), lambda i: (i, 0)),
            pl.BlockSpec((NUM_ATOM_TYPES, HIDDEN), lambda i: (0, 0)),
            pl.BlockSpec((HIDDEN, HIDDEN), lambda i: (0, 0)),
            pl.BlockSpec((1, HIDDEN), lambda i: (0, 0)),
            pl.BlockSpec((HIDDEN, 1), lambda i: (0, 0)),
            pl.BlockSpec((1, 1), lambda i: (0, 0)),
        ],
        out_specs=pl.BlockSpec((SEG_BLK, 1), lambda i: (i, 0)),
        out_shape=jax.ShapeDtypeStruct((NUM_SEGMENTS, 1), jnp.float32),
    )(h, emb, W1, b1, W2, b2)


@jax.jit
def kernel(z, batch, emb, W1, b1, W2, b2):
    batch = batch.astype(jnp.int32)
    idx = batch * NUM_ATOM_TYPES + z.astype(jnp.int32)
    idxp = jnp.concatenate(
        [idx, jnp.full((N_PADDED - N_ATOMS,), HIST_ROWS, jnp.int32)])
    bnd = jnp.searchsorted(
        batch, jnp.arange(0, NUM_SEGMENTS + 1, SEG_PER_TILE,
                          dtype=jnp.int32)).astype(jnp.int32)
    bnd = jnp.concatenate(
        [bnd, jnp.full((NBND - NW - 1,), N_ATOMS, jnp.int32)])

    hist = _sc_hist()(idxp, bnd).reshape(NUM_SEGMENTS, NUM_ATOM_TYPES)
    out = _mlp(hist, emb, W1, b1.reshape(1, HIDDEN), W2, b2.reshape(1, 1))
    return out.reshape(-1)


# speedup vs baseline: 36.9684x; 1.3290x over previous
import functools

import jax
import jax.numpy as jnp
from jax import lax
from jax.experimental import pallas as pl
from jax.experimental.pallas import tpu as pltpu
from jax.experimental.pallas import tpu_sc as plsc

NUM_SEGMENTS = 16384
NUM_ATOM_TYPES = 100
TYPE_PAD = 128
HIDDEN = 64
N_ATOMS = 800000

NUM_CORES = 2
NUM_SUBCORES = 16
NW = NUM_CORES * NUM_SUBCORES

SEG_PER_TILE = NUM_SEGMENTS // NW
HIST_TILE = SEG_PER_TILE * TYPE_PAD

IBUF = 12800
N_PADDED = N_ATOMS + IBUF
NBND = 48
PAD_IDX = NUM_SEGMENTS * TYPE_PAD


def _sc_hist_body(idx_hbm, bnd_hbm, out_hbm, ibuf, bndvec, hist):
    c = lax.axis_index("c")
    s = lax.axis_index("s")
    wid = c * NUM_SUBCORES + s
    lo_flat = wid * HIST_TILE
    hi_flat = lo_flat + HIST_TILE

    zeros = jnp.zeros((16,), jnp.float32)

    @pl.loop(0, SEG_PER_TILE)
    def _zero(i):
        for j in range(TYPE_PAD // 16):
            hist[i, pl.ds(j * 16, 16)] = zeros

    pltpu.sync_copy(bnd_hbm, bndvec)
    lanes = jax.lax.broadcasted_iota(jnp.int32, (16,), 0)

    def _bnd(w):
        vec = bndvec[pl.ds((w // 16) * 16, 16)]
        onehot = jnp.int32(1) - jnp.minimum(jnp.abs(lanes - w % 16),
                                            jnp.int32(1))
        return jnp.sum(vec * onehot)

    b_lo = _bnd(wid)
    b_hi = _bnd(wid + 1)
    start0 = (b_lo // 8) * 8
    nbatch = (b_hi - start0 + IBUF - 1) // IBUF

    one = jnp.float32(1.0)
    zero = jnp.float32(0.0)

    @pl.loop(0, nbatch)
    def _batch(t):
        base = start0 + t * IBUF
        pltpu.sync_copy(idx_hbm.at[pl.ds(base, IBUF)], ibuf)

        @plsc.parallel_loop(0, IBUF // 16, unroll=4)
        def _scan(k):
            vi = ibuf[pl.ds(k * 16, 16)]
            m = (vi >= lo_flat) & (vi < hi_flat)
            vloc = jnp.where(m, vi - lo_flat, 0)
            vseg = lax.shift_right_logical(vloc, 7)
            vtyp = vloc & (TYPE_PAD - 1)
            plsc.addupdate_scatter(hist, [vseg, vtyp],
                                   jnp.where(m, one, zero))

    pltpu.sync_copy(hist, out_hbm.at[pl.ds(wid * SEG_PER_TILE,
                                           SEG_PER_TILE)])


@functools.cache
def _sc_hist():
  return pl.kernel(
    _sc_hist_body,
    out_type=jax.ShapeDtypeStruct((NUM_SEGMENTS, TYPE_PAD), jnp.float32),
    mesh=plsc.VectorSubcoreMesh(core_axis_name="c", subcore_axis_name="s",
                                num_cores=NUM_CORES,
                                num_subcores=NUM_SUBCORES),
    compiler_params=pltpu.CompilerParams(needs_layout_passes=False),
    scratch_types=[
        pltpu.VMEM((IBUF,), jnp.int32),
        pltpu.VMEM((NBND,), jnp.int32),
        pltpu.VMEM((SEG_PER_TILE, TYPE_PAD), jnp.float32),
    ],
  )


SEG_BLK = 1024


def _mlp_body(h_ref, emb_ref, w1_ref, b1_ref, w2_ref, b2_ref, out_ref):
    hist = h_ref[...]
    counts = jnp.sum(hist, axis=1, keepdims=True)
    sums = jnp.dot(hist, emb_ref[...], preferred_element_type=jnp.float32,
                   precision=lax.Precision.HIGHEST)
    pooled = sums / jnp.maximum(counts, 1.0)
    h = jnp.maximum(
        jnp.dot(pooled, w1_ref[...], preferred_element_type=jnp.float32,
                precision=lax.Precision.HIGHEST)
        + b1_ref[...], 0.0)
    out_ref[...] = (
        jnp.dot(h, w2_ref[...], preferred_element_type=jnp.float32,
                precision=lax.Precision.HIGHEST)
        + b2_ref[...])


def _mlp(h, emb, W1, b1, W2, b2):
    return pl.pallas_call(
        _mlp_body,
        grid=(NUM_SEGMENTS // SEG_BLK,),
        in_specs=[
            pl.BlockSpec((SEG_BLK, TYPE_PAD), lambda i: (i, 0)),
            pl.BlockSpec((TYPE_PAD, HIDDEN), lambda i: (0, 0)),
            pl.BlockSpec((HIDDEN, HIDDEN), lambda i: (0, 0)),
            pl.BlockSpec((1, HIDDEN), lambda i: (0, 0)),
            pl.BlockSpec((HIDDEN, 1), lambda i: (0, 0)),
            pl.BlockSpec((1, 1), lambda i: (0, 0)),
        ],
        out_specs=pl.BlockSpec((SEG_BLK, 1), lambda i: (i, 0)),
        out_shape=jax.ShapeDtypeStruct((NUM_SEGMENTS, 1), jnp.float32),
    )(h, emb, W1, b1, W2, b2)


@jax.jit
def kernel(z, batch, emb, W1, b1, W2, b2):
    batch = batch.astype(jnp.int32)
    idx = batch * TYPE_PAD + z.astype(jnp.int32)
    idxp = jnp.concatenate(
        [idx, jnp.full((N_PADDED - N_ATOMS,), PAD_IDX, jnp.int32)])
    bnd = jnp.searchsorted(
        batch, jnp.arange(0, NUM_SEGMENTS + 1, SEG_PER_TILE,
                          dtype=jnp.int32),
        method="scan_unrolled").astype(jnp.int32)
    bnd = jnp.concatenate(
        [bnd, jnp.full((NBND - NW - 1,), N_ATOMS, jnp.int32)])

    hist = _sc_hist()(idxp, bnd)
    embp = jnp.pad(emb, ((0, TYPE_PAD - NUM_ATOM_TYPES), (0, 0)))
    out = _mlp(hist, embp, W1, b1.reshape(1, HIDDEN), W2,
               b2.reshape(1, 1))
    return out.reshape(-1)


# speedup vs baseline: 56.6520x; 1.5324x over previous
import functools

import jax
import jax.numpy as jnp
from jax import lax
from jax.experimental import pallas as pl
from jax.experimental.pallas import tpu as pltpu
from jax.experimental.pallas import tpu_sc as plsc

NUM_SEGMENTS = 16384
NUM_ATOM_TYPES = 100
TYPE_PAD = 128
HIDDEN = 64
N_ATOMS = 800000

NUM_CORES = 2
NUM_SUBCORES = 16
NW = NUM_CORES * NUM_SUBCORES

SEG_PER_TILE = NUM_SEGMENTS // NW
HIST_TILE = SEG_PER_TILE * TYPE_PAD

IBUF = 12800
N_PADDED = N_ATOMS + IBUF
NBND = 48
PAD_IDX = NUM_SEGMENTS * TYPE_PAD


def _sc_hist_body(idx_hbm, bnd_hbm, out_hbm, ibuf, bndvec, hist):
    c = lax.axis_index("c")
    s = lax.axis_index("s")
    wid = c * NUM_SUBCORES + s
    lo_flat = wid * HIST_TILE
    hi_flat = lo_flat + HIST_TILE

    zeros = jnp.zeros((16,), jnp.float32)

    @pl.loop(0, SEG_PER_TILE)
    def _zero(i):
        for j in range(TYPE_PAD // 16):
            hist[i, pl.ds(j * 16, 16)] = zeros

    pltpu.sync_copy(bnd_hbm, bndvec)
    lanes = jax.lax.broadcasted_iota(jnp.int32, (16,), 0)

    def _bnd(w):
        vec = bndvec[pl.ds((w // 16) * 16, 16)]
        onehot = jnp.int32(1) - jnp.minimum(jnp.abs(lanes - w % 16),
                                            jnp.int32(1))
        return jnp.sum(vec * onehot)

    b_lo = _bnd(wid)
    b_hi = _bnd(wid + 1)
    start0 = (b_lo // 8) * 8
    nbatch = (b_hi - start0 + IBUF - 1) // IBUF

    one = jnp.float32(1.0)
    zero = jnp.float32(0.0)

    @pl.loop(0, nbatch)
    def _batch(t):
        base = start0 + t * IBUF
        pltpu.sync_copy(idx_hbm.at[pl.ds(base, IBUF)], ibuf)

        @plsc.parallel_loop(0, IBUF // 16, unroll=4)
        def _scan(k):
            vi = ibuf[pl.ds(k * 16, 16)]
            m = (vi >= lo_flat) & (vi < hi_flat)
            vloc = jnp.where(m, vi - lo_flat, 0)
            vseg = lax.shift_right_logical(vloc, 7)
            vtyp = vloc & (TYPE_PAD - 1)
            plsc.addupdate_scatter(hist, [vseg, vtyp],
                                   jnp.where(m, one, zero))

    pltpu.sync_copy(hist, out_hbm.at[pl.ds(wid * SEG_PER_TILE,
                                           SEG_PER_TILE)])


@functools.cache
def _sc_hist():
  return pl.kernel(
    _sc_hist_body,
    out_type=jax.ShapeDtypeStruct((NUM_SEGMENTS, TYPE_PAD), jnp.float32),
    mesh=plsc.VectorSubcoreMesh(core_axis_name="c", subcore_axis_name="s",
                                num_cores=NUM_CORES,
                                num_subcores=NUM_SUBCORES),
    compiler_params=pltpu.CompilerParams(needs_layout_passes=False),
    scratch_types=[
        pltpu.VMEM((IBUF,), jnp.int32),
        pltpu.VMEM((NBND,), jnp.int32),
        pltpu.VMEM((SEG_PER_TILE, TYPE_PAD), jnp.float32),
    ],
  )


SEG_BLK = 4096


def _mlp_body(h_ref, emb_ref, w1_ref, b1_ref, w2_ref, b2_ref, out_ref):
    hist = h_ref[...]
    counts = jnp.sum(hist, axis=1, keepdims=True)
    sums = jnp.dot(hist, emb_ref[...], preferred_element_type=jnp.float32,
                   precision=lax.Precision.HIGHEST)
    pooled = sums / jnp.maximum(counts, 1.0)
    h = jnp.maximum(
        jnp.dot(pooled, w1_ref[...], preferred_element_type=jnp.float32,
                precision=lax.Precision.HIGHEST)
        + b1_ref[...], 0.0)
    out_ref[...] = (
        jnp.dot(h, w2_ref[...], preferred_element_type=jnp.float32,
                precision=lax.Precision.HIGHEST)
        + b2_ref[...])


def _mlp(h, emb, W1, b1, W2, b2):
    return pl.pallas_call(
        _mlp_body,
        grid=(NUM_SEGMENTS // SEG_BLK,),
        in_specs=[
            pl.BlockSpec((SEG_BLK, TYPE_PAD), lambda i: (i, 0)),
            pl.BlockSpec((TYPE_PAD, HIDDEN), lambda i: (0, 0)),
            pl.BlockSpec((HIDDEN, HIDDEN), lambda i: (0, 0)),
            pl.BlockSpec((1, HIDDEN), lambda i: (0, 0)),
            pl.BlockSpec((HIDDEN, 1), lambda i: (0, 0)),
            pl.BlockSpec((1, 1), lambda i: (0, 0)),
        ],
        out_specs=pl.BlockSpec((SEG_BLK, 1), lambda i: (i, 0)),
        out_shape=jax.ShapeDtypeStruct((NUM_SEGMENTS, 1), jnp.float32),
    )(h, emb, W1, b1, W2, b2)


@jax.jit
def kernel(z, batch, emb, W1, b1, W2, b2):
    batch = batch.astype(jnp.int32)
    idx = batch * TYPE_PAD + z.astype(jnp.int32)
    idxp = jnp.concatenate(
        [idx, jnp.full((N_PADDED - N_ATOMS,), PAD_IDX, jnp.int32)])
    bnd = jnp.searchsorted(
        batch, jnp.arange(0, NUM_SEGMENTS + 1, SEG_PER_TILE,
                          dtype=jnp.int32),
        method="compare_all").astype(jnp.int32)
    bnd = jnp.concatenate(
        [bnd, jnp.full((NBND - NW - 1,), N_ATOMS, jnp.int32)])

    hist = _sc_hist()(idxp, bnd)
    embp = jnp.pad(emb, ((0, TYPE_PAD - NUM_ATOM_TYPES), (0, 0)))
    out = _mlp(hist, embp, W1, b1.reshape(1, HIDDEN), W2,
               b2.reshape(1, 1))
    return out.reshape(-1)


# speedup vs baseline: 98.7469x; 1.7430x over previous
import functools

import jax
import jax.numpy as jnp
from jax import lax
from jax.experimental import pallas as pl
from jax.experimental.pallas import tpu as pltpu
from jax.experimental.pallas import tpu_sc as plsc

NUM_SEGMENTS = 16384
NUM_ATOM_TYPES = 100
TYPE_PAD = 128
HIDDEN = 64
N_ATOMS = 800000

NUM_CORES = 2
NUM_SUBCORES = 16
NW = NUM_CORES * NUM_SUBCORES

SEG_PER_TILE = NUM_SEGMENTS // NW
HIST_TILE = SEG_PER_TILE * TYPE_PAD

IBUF = 12800
N_PADDED = N_ATOMS + IBUF
NBND = 64
BND_STRIDE = 256
PAD_IDX = NUM_SEGMENTS * TYPE_PAD


def _sc_hist_body(idx_hbm, bnd_hbm, out_hbm, ibuf, bndvec, hist):
    c = lax.axis_index("c")
    s = lax.axis_index("s")
    wid = c * NUM_SUBCORES + s
    lo_flat = wid * HIST_TILE
    hi_flat = lo_flat + HIST_TILE

    zeros = jnp.zeros((16,), jnp.float32)

    @pl.loop(0, SEG_PER_TILE)
    def _zero(i):
        for j in range(TYPE_PAD // 16):
            hist[i, pl.ds(j * 16, 16)] = zeros

    pltpu.sync_copy(bnd_hbm, bndvec)
    lanes = jax.lax.broadcasted_iota(jnp.int32, (16,), 0)

    def _bnd(w):
        vec = bndvec[pl.ds((w // 16) * 16, 16)]
        onehot = jnp.int32(1) - jnp.minimum(jnp.abs(lanes - w % 16),
                                            jnp.int32(1))
        return jnp.sum(vec * onehot)

    b_lo = _bnd(wid)
    b_hi = _bnd(NW + wid)
    start0 = (b_lo // 8) * 8
    nbatch = (b_hi - start0 + IBUF - 1) // IBUF

    one = jnp.float32(1.0)
    zero = jnp.float32(0.0)

    @pl.loop(0, nbatch)
    def _batch(t):
        base = start0 + t * IBUF
        pltpu.sync_copy(idx_hbm.at[pl.ds(base, IBUF)], ibuf)

        @plsc.parallel_loop(0, IBUF // 16, unroll=4)
        def _scan(k):
            vi = ibuf[pl.ds(k * 16, 16)]
            m = (vi >= lo_flat) & (vi < hi_flat)
            vloc = jnp.where(m, vi - lo_flat, 0)
            vseg = lax.shift_right_logical(vloc, 7)
            vtyp = vloc & (TYPE_PAD - 1)
            plsc.addupdate_scatter(hist, [vseg, vtyp],
                                   jnp.where(m, one, zero))

    pltpu.sync_copy(hist, out_hbm.at[pl.ds(wid * SEG_PER_TILE,
                                           SEG_PER_TILE)])


@functools.cache
def _sc_hist():
  return pl.kernel(
    _sc_hist_body,
    out_type=jax.ShapeDtypeStruct((NUM_SEGMENTS, TYPE_PAD), jnp.float32),
    mesh=plsc.VectorSubcoreMesh(core_axis_name="c", subcore_axis_name="s",
                                num_cores=NUM_CORES,
                                num_subcores=NUM_SUBCORES),
    compiler_params=pltpu.CompilerParams(needs_layout_passes=False),
    scratch_types=[
        pltpu.VMEM((IBUF,), jnp.int32),
        pltpu.VMEM((NBND,), jnp.int32),
        pltpu.VMEM((SEG_PER_TILE, TYPE_PAD), jnp.float32),
    ],
  )


SEG_BLK = 4096


def _mlp_body(h_ref, emb_ref, w1_ref, b1_ref, w2_ref, b2_ref, out_ref):
    hist = h_ref[...]
    counts = jnp.sum(hist, axis=1, keepdims=True)
    sums = jnp.dot(hist, emb_ref[...], preferred_element_type=jnp.float32)
    pooled = sums / jnp.maximum(counts, 1.0)
    h = jnp.maximum(
        jnp.dot(pooled, w1_ref[...], preferred_element_type=jnp.float32)
        + b1_ref[...], 0.0)
    out_ref[...] = (
        jnp.dot(h, w2_ref[...], preferred_element_type=jnp.float32)
        + b2_ref[...])


def _mlp(h, emb, W1, b1, W2, b2):
    return pl.pallas_call(
        _mlp_body,
        grid=(NUM_SEGMENTS // SEG_BLK,),
        in_specs=[
            pl.BlockSpec((SEG_BLK, TYPE_PAD), lambda i: (i, 0)),
            pl.BlockSpec((TYPE_PAD, HIDDEN), lambda i: (0, 0)),
            pl.BlockSpec((HIDDEN, HIDDEN), lambda i: (0, 0)),
            pl.BlockSpec((1, HIDDEN), lambda i: (0, 0)),
            pl.BlockSpec((HIDDEN, 1), lambda i: (0, 0)),
            pl.BlockSpec((1, 1), lambda i: (0, 0)),
        ],
        out_specs=pl.BlockSpec((SEG_BLK, 1), lambda i: (i, 0)),
        out_shape=jax.ShapeDtypeStruct((NUM_SEGMENTS, 1), jnp.float32),
    )(h, emb, W1, b1, W2, b2)


@jax.jit
def kernel(z, batch, emb, W1, b1, W2, b2):
    batch = batch.astype(jnp.int32)
    idx = batch * TYPE_PAD + z.astype(jnp.int32)
    idxp = jnp.concatenate(
        [idx, jnp.full((N_PADDED - N_ATOMS,), PAD_IDX, jnp.int32)])
    sample = batch[::BND_STRIDE]
    pos = jnp.searchsorted(
        sample, jnp.arange(0, NUM_SEGMENTS + 1, SEG_PER_TILE,
                           dtype=jnp.int32),
        method="compare_all").astype(jnp.int32)
    lo = jnp.maximum(pos - 1, 0) * BND_STRIDE
    hi = jnp.minimum(pos * BND_STRIDE, N_ATOMS)
    bnd = jnp.concatenate([lo[:NW], hi[1:NW + 1]])

    hist = _sc_hist()(idxp, bnd)
    embp = jnp.pad(emb, ((0, TYPE_PAD - NUM_ATOM_TYPES), (0, 0)))
    out = _mlp(hist, embp, W1, b1.reshape(1, HIDDEN), W2,
               b2.reshape(1, 1))
    return out.reshape(-1)


# speedup vs baseline: 102.6545x; 1.0396x over previous
import functools

import jax
import jax.numpy as jnp
from jax import lax
from jax.experimental import pallas as pl
from jax.experimental.pallas import tpu as pltpu
from jax.experimental.pallas import tpu_sc as plsc

NUM_SEGMENTS = 16384
NUM_ATOM_TYPES = 100
TYPE_PAD = 128
HIDDEN = 64
N_ATOMS = 800000

NUM_CORES = 2
NUM_SUBCORES = 16
NW = NUM_CORES * NUM_SUBCORES

SEG_PER_TILE = NUM_SEGMENTS // NW
HIST_TILE = SEG_PER_TILE * TYPE_PAD

IBUF = 12800
N_PADDED = N_ATOMS + IBUF
NBND = 64
BND_STRIDE = 256
PAD_IDX = NUM_SEGMENTS * TYPE_PAD


def _sc_hist_body(idx_hbm, bnd_hbm, out_hbm, ibuf, bndvec, hist):
    c = lax.axis_index("c")
    s = lax.axis_index("s")
    wid = c * NUM_SUBCORES + s
    lo_flat = wid * HIST_TILE
    hi_flat = lo_flat + HIST_TILE

    zeros = jnp.zeros((16,), jnp.float32)

    @pl.loop(0, SEG_PER_TILE)
    def _zero(i):
        for j in range(TYPE_PAD // 16):
            hist[i, pl.ds(j * 16, 16)] = zeros

    pltpu.sync_copy(bnd_hbm, bndvec)
    lanes = jax.lax.broadcasted_iota(jnp.int32, (16,), 0)

    def _bnd(w):
        vec = bndvec[pl.ds((w // 16) * 16, 16)]
        onehot = jnp.int32(1) - jnp.minimum(jnp.abs(lanes - w % 16),
                                            jnp.int32(1))
        return jnp.sum(vec * onehot)

    b_lo = _bnd(wid)
    b_hi = _bnd(NW + wid)
    start0 = (b_lo // 8) * 8
    nbatch = (b_hi - start0 + IBUF - 1) // IBUF

    one = jnp.float32(1.0)
    zero = jnp.float32(0.0)

    @pl.loop(0, nbatch)
    def _batch(t):
        base = start0 + t * IBUF
        pltpu.sync_copy(idx_hbm.at[pl.ds(base, IBUF)], ibuf)

        @plsc.parallel_loop(0, IBUF // 16, unroll=4)
        def _scan(k):
            vi = ibuf[pl.ds(k * 16, 16)]
            m = (vi >= lo_flat) & (vi < hi_flat)
            vloc = jnp.where(m, vi - lo_flat, 0)
            vseg = lax.shift_right_logical(vloc, 7)
            vtyp = vloc & (TYPE_PAD - 1)
            plsc.addupdate_scatter(hist, [vseg, vtyp],
                                   jnp.where(m, one, zero))

    pltpu.sync_copy(hist, out_hbm.at[pl.ds(wid * SEG_PER_TILE,
                                           SEG_PER_TILE)])


@functools.cache
def _sc_hist():
  return pl.kernel(
    _sc_hist_body,
    out_type=jax.ShapeDtypeStruct((NUM_SEGMENTS, TYPE_PAD), jnp.float32),
    mesh=plsc.VectorSubcoreMesh(core_axis_name="c", subcore_axis_name="s",
                                num_cores=NUM_CORES,
                                num_subcores=NUM_SUBCORES),
    compiler_params=pltpu.CompilerParams(needs_layout_passes=False),
    scratch_types=[
        pltpu.VMEM((IBUF,), jnp.int32),
        pltpu.VMEM((NBND,), jnp.int32),
        pltpu.VMEM((SEG_PER_TILE, TYPE_PAD), jnp.float32),
    ],
  )


SEG_BLK = 4096


def _mlp_body(h_ref, emb_ref, w1_ref, b1_ref, w2_ref, b2_ref, out_ref):
    hist = h_ref[...]
    counts = jnp.sum(hist, axis=1, keepdims=True)
    sums = jnp.dot(hist, emb_ref[...], preferred_element_type=jnp.float32)
    pooled = sums / jnp.maximum(counts, 1.0)
    h = jnp.maximum(
        jnp.dot(pooled, w1_ref[...], preferred_element_type=jnp.float32)
        + b1_ref[...], 0.0)
    out = (jnp.dot(h, w2_ref[...], preferred_element_type=jnp.float32)
           + b2_ref[...])
    out_ref[...] = out[:, 0]


def _mlp(h, emb, W1, b1, W2, b2):
    return pl.pallas_call(
        _mlp_body,
        grid=(NUM_SEGMENTS // SEG_BLK,),
        in_specs=[
            pl.BlockSpec((SEG_BLK, TYPE_PAD), lambda i: (i, 0)),
            pl.BlockSpec((TYPE_PAD, HIDDEN), lambda i: (0, 0)),
            pl.BlockSpec((HIDDEN, HIDDEN), lambda i: (0, 0)),
            pl.BlockSpec((1, HIDDEN), lambda i: (0, 0)),
            pl.BlockSpec((HIDDEN, 1), lambda i: (0, 0)),
            pl.BlockSpec((1, 1), lambda i: (0, 0)),
        ],
        out_specs=pl.BlockSpec((SEG_BLK,), lambda i: (i,)),
        out_shape=jax.ShapeDtypeStruct((NUM_SEGMENTS,), jnp.float32),
    )(h, emb, W1, b1, W2, b2)


@jax.jit
def kernel(z, batch, emb, W1, b1, W2, b2):
    batch = batch.astype(jnp.int32)
    idx = batch * TYPE_PAD + z.astype(jnp.int32)
    idxp = jnp.concatenate(
        [idx, jnp.full((N_PADDED - N_ATOMS,), PAD_IDX, jnp.int32)])
    sample = batch[::BND_STRIDE]
    pos = jnp.searchsorted(
        sample, jnp.arange(0, NUM_SEGMENTS + 1, SEG_PER_TILE,
                           dtype=jnp.int32),
        method="compare_all").astype(jnp.int32)
    lo = jnp.maximum(pos - 1, 0) * BND_STRIDE
    hi = jnp.minimum(pos * BND_STRIDE, N_ATOMS)
    bnd = jnp.concatenate([lo[:NW], hi[1:NW + 1]])

    hist = _sc_hist()(idxp, bnd)
    embp = jnp.pad(emb, ((0, TYPE_PAD - NUM_ATOM_TYPES), (0, 0)))
    return _mlp(hist, embp, W1, b1.reshape(1, HIDDEN), W2,
                b2.reshape(1, 1))


# speedup vs baseline: 106.8851x; 1.0412x over previous
import functools

import jax
import jax.numpy as jnp
from jax import lax
from jax.experimental import pallas as pl
from jax.experimental.pallas import tpu as pltpu
from jax.experimental.pallas import tpu_sc as plsc

NUM_SEGMENTS = 16384
NUM_ATOM_TYPES = 100
TYPE_PAD = 128
HIDDEN = 64
N_ATOMS = 800000

NUM_CORES = 2
NUM_SUBCORES = 16
NW = NUM_CORES * NUM_SUBCORES

SEG_PER_TILE = NUM_SEGMENTS // NW
HIST_TILE = SEG_PER_TILE * TYPE_PAD

IBUF = 12800
N_PADDED = N_ATOMS + IBUF
NBND = 64
BND_STRIDE = 256
PAD_IDX = NUM_SEGMENTS * TYPE_PAD


def _sc_hist_body(idx_hbm, bnd_hbm, out_hbm, ibuf0, ibuf1, bndvec, hist,
                  sem):
    c = lax.axis_index("c")
    s = lax.axis_index("s")
    wid = c * NUM_SUBCORES + s
    lo_flat = wid * HIST_TILE
    hi_flat = lo_flat + HIST_TILE

    pltpu.sync_copy(bnd_hbm, bndvec)
    lanes = jax.lax.broadcasted_iota(jnp.int32, (16,), 0)

    def _bnd(w):
        vec = bndvec[pl.ds((w // 16) * 16, 16)]
        onehot = jnp.int32(1) - jnp.minimum(jnp.abs(lanes - w % 16),
                                            jnp.int32(1))
        return jnp.sum(vec * onehot)

    b_lo = _bnd(wid)
    b_hi = _bnd(NW + wid)
    start0 = (b_lo // 8) * 8
    nbatch = (b_hi - start0 + IBUF - 1) // IBUF

    pltpu.async_copy(idx_hbm.at[pl.ds(start0, IBUF)], ibuf0, sem)

    zeros = jnp.zeros((16,), jnp.float32)

    @pl.loop(0, SEG_PER_TILE)
    def _zero(i):
        for j in range(TYPE_PAD // 16):
            hist[i, pl.ds(j * 16, 16)] = zeros

    one = jnp.float32(1.0)
    zero = jnp.float32(0.0)

    def _step(t, cur, nxt):
        pltpu.make_async_copy(idx_hbm.at[pl.ds(start0, IBUF)],
                              cur, sem).wait()

        @pl.when(t + 1 < nbatch)
        def _prefetch():
            base = start0 + (t + 1) * IBUF
            pltpu.async_copy(idx_hbm.at[pl.ds(base, IBUF)], nxt, sem)

        @plsc.parallel_loop(0, IBUF // 16, unroll=4)
        def _scan(k):
            vi = cur[pl.ds(k * 16, 16)]
            m = (vi >= lo_flat) & (vi < hi_flat)
            vloc = jnp.where(m, vi - lo_flat, 0)
            vseg = lax.shift_right_logical(vloc, 7)
            vtyp = vloc & (TYPE_PAD - 1)
            plsc.addupdate_scatter(hist, [vseg, vtyp],
                                   jnp.where(m, one, zero))

    @pl.loop(0, nbatch)
    def _batch(t):
        @pl.when(t % 2 == 0)
        def _even():
            _step(t, ibuf0, ibuf1)

        @pl.when(t % 2 == 1)
        def _odd():
            _step(t, ibuf1, ibuf0)

    pltpu.sync_copy(hist, out_hbm.at[pl.ds(wid * SEG_PER_TILE,
                                           SEG_PER_TILE)])


@functools.cache
def _sc_hist():
  return pl.kernel(
    _sc_hist_body,
    out_type=jax.ShapeDtypeStruct((NUM_SEGMENTS, TYPE_PAD), jnp.float32),
    mesh=plsc.VectorSubcoreMesh(core_axis_name="c", subcore_axis_name="s",
                                num_cores=NUM_CORES,
                                num_subcores=NUM_SUBCORES),
    compiler_params=pltpu.CompilerParams(needs_layout_passes=False),
    scratch_types=[
        pltpu.VMEM((IBUF,), jnp.int32),
        pltpu.VMEM((IBUF,), jnp.int32),
        pltpu.VMEM((NBND,), jnp.int32),
        pltpu.VMEM((SEG_PER_TILE, TYPE_PAD), jnp.float32),
        pltpu.SemaphoreType.DMA,
    ],
  )


SEG_BLK = 4096


def _mlp_body(h_ref, emb_ref, w1_ref, b1_ref, w2_ref, b2_ref, out_ref):
    hist = h_ref[...]
    counts = jnp.sum(hist, axis=1, keepdims=True)
    sums = jnp.dot(hist, emb_ref[...], preferred_element_type=jnp.float32)
    pooled = sums / jnp.maximum(counts, 1.0)
    h = jnp.maximum(
        jnp.dot(pooled, w1_ref[...], preferred_element_type=jnp.float32)
        + b1_ref[...], 0.0)
    out = (jnp.dot(h, w2_ref[...], preferred_element_type=jnp.float32)
           + b2_ref[...])
    out_ref[...] = out[:, 0]


def _mlp(h, emb, W1, b1, W2, b2):
    return pl.pallas_call(
        _mlp_body,
        grid=(NUM_SEGMENTS // SEG_BLK,),
        in_specs=[
            pl.BlockSpec((SEG_BLK, TYPE_PAD), lambda i: (i, 0)),
            pl.BlockSpec((TYPE_PAD, HIDDEN), lambda i: (0, 0)),
            pl.BlockSpec((HIDDEN, HIDDEN), lambda i: (0, 0)),
            pl.BlockSpec((1, HIDDEN), lambda i: (0, 0)),
            pl.BlockSpec((HIDDEN, 1), lambda i: (0, 0)),
            pl.BlockSpec((1, 1), lambda i: (0, 0)),
        ],
        out_specs=pl.BlockSpec((SEG_BLK,), lambda i: (i,)),
        out_shape=jax.ShapeDtypeStruct((NUM_SEGMENTS,), jnp.float32),
    )(h, emb, W1, b1, W2, b2)


@jax.jit
def kernel(z, batch, emb, W1, b1, W2, b2):
    batch = batch.astype(jnp.int32)
    idx = batch * TYPE_PAD + z.astype(jnp.int32)
    idxp = jnp.concatenate(
        [idx, jnp.full((N_PADDED - N_ATOMS,), PAD_IDX, jnp.int32)])
    sample = batch[::BND_STRIDE]
    pos = jnp.searchsorted(
        sample, jnp.arange(0, NUM_SEGMENTS + 1, SEG_PER_TILE,
                           dtype=jnp.int32),
        method="compare_all").astype(jnp.int32)
    lo = jnp.maximum(pos - 1, 0) * BND_STRIDE
    hi = jnp.minimum(pos * BND_STRIDE, N_ATOMS)
    bnd = jnp.concatenate([lo[:NW], hi[1:NW + 1]])

    hist = _sc_hist()(idxp, bnd)
    embp = jnp.pad(emb, ((0, TYPE_PAD - NUM_ATOM_TYPES), (0, 0)))
    return _mlp(hist, embp, W1, b1.reshape(1, HIDDEN), W2,
                b2.reshape(1, 1))


# speedup vs baseline: 116.7845x; 1.0926x over previous
import functools

import jax
import jax.numpy as jnp
from jax import lax
from jax.experimental import pallas as pl
from jax.experimental.pallas import tpu as pltpu
from jax.experimental.pallas import tpu_sc as plsc

NUM_SEGMENTS = 16384
NUM_ATOM_TYPES = 100
TYPE_PAD = 128
HIDDEN = 64
N_ATOMS = 800000

NUM_CORES = 2
NUM_SUBCORES = 16
NW = NUM_CORES * NUM_SUBCORES

SEG_PER_TILE = NUM_SEGMENTS // NW
HIST_TILE = SEG_PER_TILE * TYPE_PAD

IBUF = 12800
N_PADDED = N_ATOMS + IBUF
NBND = 64
BND_STRIDE = 256
PAD_IDX = NUM_SEGMENTS * TYPE_PAD


def _sc_hist_body(z_hbm, b_hbm, bnd_hbm, out_hbm, zb0, zb1, bb0, bb1,
                  bndvec, hist, sem):
    c = lax.axis_index("c")
    s = lax.axis_index("s")
    wid = c * NUM_SUBCORES + s
    lo_seg = wid * SEG_PER_TILE
    hi_seg = lo_seg + SEG_PER_TILE

    pltpu.sync_copy(bnd_hbm, bndvec)
    lanes = jax.lax.broadcasted_iota(jnp.int32, (16,), 0)

    def _bnd(w):
        vec = bndvec[pl.ds((w // 16) * 16, 16)]
        onehot = jnp.int32(1) - jnp.minimum(jnp.abs(lanes - w % 16),
                                            jnp.int32(1))
        return jnp.sum(vec * onehot)

    b_lo = _bnd(wid)
    b_hi = _bnd(NW + wid)
    start0 = (b_lo // 8) * 8
    nbatch = (b_hi - start0 + IBUF - 1) // IBUF

    def _base(t):
        bu = start0 + t * IBUF
        return jnp.minimum(bu, N_ATOMS - IBUF), bu

    def _fire(t, zb, bb):
        base, _ = _base(t)
        pltpu.async_copy(z_hbm.at[pl.ds(base, IBUF)], zb, sem)
        pltpu.async_copy(b_hbm.at[pl.ds(base, IBUF)], bb, sem)

    @pl.when(nbatch > 0)
    def _fire0():
        _fire(0, zb0, bb0)

    zeros = jnp.zeros((16,), jnp.float32)

    @pl.loop(0, SEG_PER_TILE)
    def _zero(i):
        for j in range(TYPE_PAD // 16):
            hist[i, pl.ds(j * 16, 16)] = zeros

    one = jnp.float32(1.0)
    zero = jnp.float32(0.0)

    def _step(t, zcur, bcur, znxt, bnxt):
        base, bu = _base(t)
        off = bu - base
        pltpu.make_async_copy(z_hbm.at[pl.ds(base, IBUF)], zcur,
                              sem).wait()
        pltpu.make_async_copy(b_hbm.at[pl.ds(base, IBUF)], bcur,
                              sem).wait()

        @pl.when(t + 1 < nbatch)
        def _prefetch():
            _fire(t + 1, znxt, bnxt)

        @plsc.parallel_loop(0, IBUF // 16, unroll=4)
        def _scan(k):
            vz = zcur[pl.ds(k * 16, 16)]
            vb = bcur[pl.ds(k * 16, 16)]
            m = (vb >= lo_seg) & (vb < hi_seg) & (lanes >= off - k * 16)
            vseg = jnp.where(m, vb - lo_seg, 0)
            plsc.addupdate_scatter(hist, [vseg, vz],
                                   jnp.where(m, one, zero))

    @pl.loop(0, nbatch)
    def _batch(t):
        @pl.when(t % 2 == 0)
        def _even():
            _step(t, zb0, bb0, zb1, bb1)

        @pl.when(t % 2 == 1)
        def _odd():
            _step(t, zb1, bb1, zb0, bb0)

    pltpu.sync_copy(hist, out_hbm.at[pl.ds(wid * SEG_PER_TILE,
                                           SEG_PER_TILE)])


@functools.cache
def _sc_hist():
  return pl.kernel(
    _sc_hist_body,
    out_type=jax.ShapeDtypeStruct((NUM_SEGMENTS, TYPE_PAD), jnp.float32),
    mesh=plsc.VectorSubcoreMesh(core_axis_name="c", subcore_axis_name="s",
                                num_cores=NUM_CORES,
                                num_subcores=NUM_SUBCORES),
    compiler_params=pltpu.CompilerParams(needs_layout_passes=False),
    scratch_types=[
        pltpu.VMEM((IBUF,), jnp.int32),
        pltpu.VMEM((IBUF,), jnp.int32),
        pltpu.VMEM((IBUF,), jnp.int32),
        pltpu.VMEM((IBUF,), jnp.int32),
        pltpu.VMEM((NBND,), jnp.int32),
        pltpu.VMEM((SEG_PER_TILE, TYPE_PAD), jnp.float32),
        pltpu.SemaphoreType.DMA,
    ],
  )


SEG_BLK = 4096


def _mlp_body(h_ref, emb_ref, w1_ref, b1_ref, w2_ref, b2_ref, out_ref):
    hist = h_ref[...]
    counts = jnp.sum(hist, axis=1, keepdims=True)
    sums = jnp.dot(hist, emb_ref[...], preferred_element_type=jnp.float32)
    pooled = sums / jnp.maximum(counts, 1.0)
    h = jnp.maximum(
        jnp.dot(pooled, w1_ref[...], preferred_element_type=jnp.float32)
        + b1_ref[...], 0.0)
    out = (jnp.dot(h, w2_ref[...], preferred_element_type=jnp.float32)
           + b2_ref[...])
    out_ref[...] = out[:, 0]


def _mlp(h, emb, W1, b1, W2, b2):
    return pl.pallas_call(
        _mlp_body,
        grid=(NUM_SEGMENTS // SEG_BLK,),
        in_specs=[
            pl.BlockSpec((SEG_BLK, TYPE_PAD), lambda i: (i, 0)),
            pl.BlockSpec((TYPE_PAD, HIDDEN), lambda i: (0, 0)),
            pl.BlockSpec((HIDDEN, HIDDEN), lambda i: (0, 0)),
            pl.BlockSpec((1, HIDDEN), lambda i: (0, 0)),
            pl.BlockSpec((HIDDEN, 1), lambda i: (0, 0)),
            pl.BlockSpec((1, 1), lambda i: (0, 0)),
        ],
        out_specs=pl.BlockSpec((SEG_BLK,), lambda i: (i,)),
        out_shape=jax.ShapeDtypeStruct((NUM_SEGMENTS,), jnp.float32),
    )(h, emb, W1, b1, W2, b2)


@jax.jit
def kernel(z, batch, emb, W1, b1, W2, b2):
    z = z.astype(jnp.int32)
    batch = batch.astype(jnp.int32)
    sample = batch[::BND_STRIDE]
    pos = jnp.searchsorted(
        sample, jnp.arange(0, NUM_SEGMENTS + 1, SEG_PER_TILE,
                           dtype=jnp.int32),
        method="compare_all").astype(jnp.int32)
    lo = jnp.maximum(pos - 1, 0) * BND_STRIDE
    hi = jnp.minimum(pos * BND_STRIDE, N_ATOMS)
    bnd = jnp.concatenate([lo[:NW], hi[1:NW + 1]])

    hist = _sc_hist()(z, batch, bnd)
    embp = jnp.pad(emb, ((0, TYPE_PAD - NUM_ATOM_TYPES), (0, 0)))
    return _mlp(hist, embp, W1, b1.reshape(1, HIDDEN), W2,
                b2.reshape(1, 1))
